# Initial kernel scaffold; baseline (speedup 1.0000x reference)
import jax
import jax.numpy as jnp
from jax.experimental import pallas as pl


def kernel(hidden_state, start_indices, end_indices, missing_embeddings):
    raise NotImplementedError("write your pallas kernel here")


# speedup vs baseline: 1.6519x; 1.6519x over previous
import functools

import jax
import jax.numpy as jnp
from jax import lax
from jax.experimental import pallas as pl
from jax.experimental.pallas import tpu as pltpu
from jax.experimental.pallas import tpu_sc as plsc

_NC = 2
_NS = 16
_L = 16
_NEG = float(jnp.finfo(jnp.float32).min)
_I32_MIN = -(2 ** 31)


@functools.lru_cache(maxsize=None)
def _build(B, S, H, NI):
    NW = _NC * _NS
    P = B * NI
    assert P % _L == 0 and H % _L == 0
    PPW = (P + NW - 1) // NW
    HC = H // _L

    mesh = plsc.VectorSubcoreMesh(core_axis_name="c", subcore_axis_name="s")

    @functools.partial(
        pl.kernel,
        mesh=mesh,
        out_type=jax.ShapeDtypeStruct((P, H), jnp.float32),
        scratch_types=[
            pltpu.VMEM((_L, H), jnp.float32),
            pltpu.VMEM((H,), jnp.float32),
            pltpu.VMEM((P + _L,), jnp.int32),
            pltpu.VMEM((P + _L,), jnp.int32),
            pltpu.SemaphoreType.DMA,
        ],
    )
    def sc_kernel(hid_hbm, start_hbm, end_hbm, miss_hbm, out_hbm,
                  rows_v, acc_v, s_v, e_v, sem):
        wid = lax.axis_index("s") * _NC + lax.axis_index("c")
        lanes = lax.iota(jnp.int32, _L)

        def do_pair(p):
            b = p // NI
            i = p % NI

            pltpu.sync_copy(start_hbm, s_v.at[pl.ds(0, P)])
            pltpu.sync_copy(end_hbm, e_v.at[pl.ds(0, P)])
            s_v[pl.ds(P, _L)] = jnp.zeros((_L,), jnp.int32)
            e_v[pl.ds(P, _L)] = jnp.zeros((_L,), jnp.int32)
            s = s_v[pl.ds(p, _L)][0]
            e = e_v[pl.ds(p, _L)][0]
            valid = jnp.logical_and(s >= 0, e >= 0)
            cs = jnp.clip(s, 0, S)
            ce = jnp.clip(e, 0, S)
            ln = ce - cs
            base = b * S + cs

            for hc in range(HC):
                acc_v[pl.ds(hc * _L, _L)] = jnp.full((_L,), _NEG, jnp.float32)

            nchunks = jnp.maximum((ln + _L - 1) // _L, 0)

            def chunk_body(c, carry):
                roff = c * _L + lanes
                idx = base + jnp.minimum(roff, ln - 1)
                pltpu.async_copy(hid_hbm.at[idx], rows_v, sem).wait()
                nr = jnp.minimum(ln - c * _L, _L)

                def row_body(r, rc):
                    for hc in range(HC):
                        sl = pl.ds(hc * _L, _L)
                        acc_v[sl] = jnp.maximum(acc_v[sl], rows_v[r, sl])
                    return rc

                return lax.fori_loop(0, nr, row_body, carry)

            lax.fori_loop(0, nchunks, chunk_body, jnp.int32(0))

            @pl.when(jnp.logical_not(valid))
            def _():
                pltpu.sync_copy(miss_hbm.at[i], acc_v)

            pltpu.sync_copy(acc_v, out_hbm.at[p])

        for t in range(PPW):
            p = wid + t * NW
            if P % NW == 0:
                do_pair(p)
            else:
                pl.when(p < P)(lambda: do_pair(p))

    return sc_kernel


def kernel(hidden_state, start_indices, end_indices, missing_embeddings):
    B, S, H = hidden_state.shape
    NI = start_indices.shape[1]
    sc = _build(B, S, H, NI)
    out = sc(
        hidden_state.reshape(B * S, H),
        start_indices.reshape(B * NI),
        end_indices.reshape(B * NI),
        missing_embeddings,
    )
    return out.reshape(B, NI * H)
